# Initial kernel scaffold; baseline (speedup 1.0000x reference)
import jax
import jax.numpy as jnp
from jax.experimental import pallas as pl


def kernel(scores, bbox_deltas, image_width, image_height, is_training):
    raise NotImplementedError("write your pallas kernel here")


# speedup vs baseline: 184.0196x; 184.0196x over previous
import numpy as np
import jax
import jax.numpy as jnp
from jax import lax
from jax.experimental import pallas as pl
from jax.experimental.pallas import tpu as pltpu

_FEAT_STRIDE = 16
_PRE_NMS = 6000
_POST_NMS = 300
_NMS_THRESH = 0.7
_N = 9 * 64 * 64
_ROWS, _COLS = 8, 4608
_NEG = -jnp.inf


def _gen_anchors():
    base = np.array([1, 1, _FEAT_STRIDE, _FEAT_STRIDE], dtype=np.float64) - 1
    w = base[2] - base[0] + 1.0
    h = base[3] - base[1] + 1.0
    xc, yc = base[0] + 0.5 * (w - 1), base[1] + 0.5 * (h - 1)
    ratios = np.array([0.5, 1.0, 2.0])
    size = w * h
    ws_r = np.round(np.sqrt(size / ratios))
    hs_r = np.round(ws_r * ratios)
    anchors = []
    for wr, hr in zip(ws_r, hs_r):
        xcr = xc
        ycr = yc
        for s in (8.0, 16.0, 32.0):
            wss, hss = wr * s, hr * s
            anchors.append([xcr - 0.5 * (wss - 1), ycr - 0.5 * (hss - 1),
                            xcr + 0.5 * (wss - 1), ycr + 0.5 * (hss - 1)])
    return np.array(anchors, dtype=np.float64)


def _anchor_planes():
    a = _gen_anchors()
    sx = (np.arange(64, dtype=np.float64) * _FEAT_STRIDE)
    sy = (np.arange(64, dtype=np.float64) * _FEAT_STRIDE)
    SX, SY = np.meshgrid(sx, sy)
    shift = np.stack([SX.ravel(), SY.ravel(), SX.ravel(), SY.ravel()], axis=1)
    full = a[None, :, :] + shift[:, None, :]
    full = full.reshape(_N, 4)
    x1, y1, x2, y2 = full[:, 0], full[:, 1], full[:, 2], full[:, 3]
    W = x2 - x1 + 1.0
    H = y2 - y1 + 1.0
    CX = x1 + 0.5 * W
    CY = y1 + 0.5 * H
    packs = [p.reshape(_ROWS, _COLS).astype(np.float32) for p in (W, H, CX, CY)]
    return [jnp.asarray(p) for p in packs]


_AW, _AH, _ACX, _ACY = _anchor_planes()


def _proposal_kernel(sc_ref, dx_ref, dy_ref, dw_ref, dh_ref,
                     aw_ref, ah_ref, acx_ref, acy_ref, bnd_ref,
                     out_ref,
                     x1_ref, y1_ref, x2_ref, y2_ref, ar_ref, ms_ref, idx_ref):
    b = pl.program_id(0)

    W = aw_ref[...]
    H = ah_ref[...]
    pcx = dx_ref[0] * W + acx_ref[...]
    pcy = dy_ref[0] * H + acy_ref[...]
    pw = jnp.exp(dw_ref[0]) * W
    ph = jnp.exp(dh_ref[0]) * H
    mw = bnd_ref[0, 0] - 1.0
    mh = bnd_ref[0, 1] - 1.0
    x1 = jnp.clip(pcx - 0.5 * pw, 0.0, mw)
    y1 = jnp.clip(pcy - 0.5 * ph, 0.0, mh)
    x2 = jnp.clip(pcx + 0.5 * pw, 0.0, mw)
    y2 = jnp.clip(pcy + 0.5 * ph, 0.0, mh)
    x1_ref[...] = x1
    y1_ref[...] = y1
    x2_ref[...] = x2
    y2_ref[...] = y2
    ar_ref[...] = (x2 - x1 + 1.0) * (y2 - y1 + 1.0)

    ri = lax.broadcasted_iota(jnp.int32, (_ROWS, _COLS), 0)
    ci = lax.broadcasted_iota(jnp.int32, (_ROWS, _COLS), 1)
    idx = ri * _COLS + ci
    idx_ref[...] = idx

    s = sc_ref[0]
    si = lax.bitcast_convert_type(s, jnp.int32)
    keys = jnp.where(si < 0, si ^ jnp.int32(0x7FFFFFFF), si)

    sign = jnp.int32(-2147483648)
    cand = jnp.int32(0)
    for bbit in range(31, -1, -1):
        bit = jnp.int32(-(1 << 31)) if bbit == 31 else jnp.int32(1 << bbit)
        cand2 = cand | bit
        cnt = jnp.sum((keys >= (cand2 ^ sign)).astype(jnp.int32))
        cand = jnp.where(cnt >= _PRE_NMS, cand2, cand)
    Vs = cand ^ sign

    c_gt = jnp.sum((keys > Vs).astype(jnp.int32))
    r = jnp.int32(_PRE_NMS) - c_gt
    eq = keys == Vs
    mc = jnp.int32(0)
    for bbit in range(16, -1, -1):
        cand2 = mc | jnp.int32(1 << bbit)
        g = jnp.sum((eq & (idx < cand2)).astype(jnp.int32))
        mc = jnp.where(g <= r, cand2, mc)
    cand_mask = (keys > Vs) | (eq & (idx < mc))
    ms_ref[...] = jnp.where(cand_mask, s, _NEG)

    si8 = lax.broadcasted_iota(jnp.int32, (_ROWS, 512), 0)
    li = lax.broadcasted_iota(jnp.int32, (_ROWS, 512), 1)
    bf = b.astype(jnp.float32)
    out_ref[0] = jnp.where(si8 == 4, bf, 0.0)

    def body(i, carry):
        ms = ms_ref[...]
        m = jnp.max(ms)
        valid = m != _NEG
        is_m = ms == m
        idxv = idx_ref[...]
        selidx = jnp.min(jnp.where(is_m, idxv, jnp.int32(2147483647)))
        sel = is_m & (idxv == selidx)
        x1 = x1_ref[...]
        y1 = y1_ref[...]
        x2 = x2_ref[...]
        y2 = y2_ref[...]
        x1s = jnp.sum(jnp.where(sel, x1, 0.0))
        y1s = jnp.sum(jnp.where(sel, y1, 0.0))
        x2s = jnp.sum(jnp.where(sel, x2, 0.0))
        y2s = jnp.sum(jnp.where(sel, y2, 0.0))
        ars = (x2s - x1s + 1.0) * (y2s - y1s + 1.0)
        xx1 = jnp.maximum(x1, x1s)
        yy1 = jnp.maximum(y1, y1s)
        xx2 = jnp.minimum(x2, x2s)
        yy2 = jnp.minimum(y2, y2s)
        w = jnp.maximum(xx2 - xx1 + 1.0, 0.0)
        h = jnp.maximum(yy2 - yy1 + 1.0, 0.0)
        inter = w * h
        ar = ar_ref[...]
        sup = ((1.0 + _NMS_THRESH) * inter > _NMS_THRESH * (ars + ar)) & valid
        ms_ref[...] = jnp.where(sup, _NEG, ms)
        onehot = (li == i) & valid
        vals = jnp.where(si8 == 0, x1s,
               jnp.where(si8 == 1, y1s,
               jnp.where(si8 == 2, x2s, y2s)))
        out_ref[0] = out_ref[0] + jnp.where(onehot & (si8 < 4), vals, 0.0)
        return carry

    lax.fori_loop(0, _POST_NMS, body, jnp.int32(0))


def kernel(scores, bbox_deltas, image_width, image_height, is_training):
    bsz = scores.shape[0]
    na = 9
    sc = scores[:, na:, :, :].transpose(0, 2, 3, 1).reshape(bsz, _ROWS, _COLS)
    d = bbox_deltas.transpose(0, 2, 3, 1).reshape(bsz, _N, 4)
    dx = d[..., 0].reshape(bsz, _ROWS, _COLS)
    dy = d[..., 1].reshape(bsz, _ROWS, _COLS)
    dw = d[..., 2].reshape(bsz, _ROWS, _COLS)
    dh = d[..., 3].reshape(bsz, _ROWS, _COLS)
    bnd = jnp.stack([jnp.asarray(image_width, jnp.float32),
                     jnp.asarray(image_height, jnp.float32)]).reshape(1, 2)

    bspec = pl.BlockSpec((1, _ROWS, _COLS), lambda b: (b, 0, 0))
    cspec = pl.BlockSpec((_ROWS, _COLS), lambda b: (0, 0))
    out = pl.pallas_call(
        _proposal_kernel,
        grid=(bsz,),
        in_specs=[bspec, bspec, bspec, bspec, bspec,
                  cspec, cspec, cspec, cspec,
                  pl.BlockSpec((1, 2), lambda b: (0, 0))],
        out_specs=pl.BlockSpec((1, _ROWS, 512), lambda b: (b, 0, 0)),
        out_shape=jax.ShapeDtypeStruct((bsz, _ROWS, 512), jnp.float32),
        scratch_shapes=[pltpu.VMEM((_ROWS, _COLS), jnp.float32)] * 6
                       + [pltpu.VMEM((_ROWS, _COLS), jnp.int32)],
    )(sc, dx, dy, dw, dh, _AW, _AH, _ACX, _ACY, bnd)

    coords = out[:, 0:4, :_POST_NMS]
    col0 = out[:, 4:5, :_POST_NMS]
    return jnp.concatenate([col0, coords], axis=1).transpose(0, 2, 1)
